# Initial kernel scaffold; baseline (speedup 1.0000x reference)
import functools

import jax
import jax.numpy as jnp
from jax import lax
from jax.experimental import pallas as pl
from jax.experimental.pallas import tpu as pltpu
from jax.experimental.pallas import tpu_sc as plsc

N = 10000
E = 320000
F_IN = 128
H = 32
C = 2
EPS = 1e-15

NW = 32
NSUB = 16
CHUNK = 1024
SUB = 128
NSUBC = CHUNK // SUB
EPT = 10240
E_PAD = NW * EPT
N_CHUNKS = EPT // CHUNK
ACC_ROWS = 10240
ZROWS = ACC_ROWS // NSUB
ROWS_OUT = N // NSUB

RB = 1000
GRID = N // RB


def _sc_segment(width):
    mesh = plsc.VectorSubcoreMesh(core_axis_name="c", subcore_axis_name="s")

    def body(table, src2, dst2, ewr, prm, zrows, out,
             acc, srcv, dstv, ewv, gv, rows, prmv, sem):
        c = lax.axis_index("c")
        s = lax.axis_index("s")
        wid = c * NSUB + s
        pltpu.sync_copy(zrows, acc.at[pl.ds(s * ZROWS, ZROWS)])
        pltpu.sync_copy(prm, prmv)
        plsc.subcore_barrier()
        mu = prmv[0]
        cf = prmv[1]

        def chunk(k, carry):
            base = wid * EPT + k * CHUNK
            rbase = wid * (EPT // SUB) + k * NSUBC
            pltpu.sync_copy(src2.at[pl.ds(rbase, NSUBC)], srcv)
            pltpu.sync_copy(dst2.at[pl.ds(rbase, NSUBC)], dstv)
            pltpu.sync_copy(ewr.at[pl.ds(base, CHUNK)], ewv)
            cps = [
                pltpu.async_copy(table.at[srcv.at[j]],
                                 rows.at[pl.ds(j * SUB, SUB)], sem)
                for j in range(NSUBC)
            ]

            def gbody(j, _):
                w = ewv[pl.ds(j * 16, 16)]
                d = w - mu
                gv[pl.ds(j * 16, 16)] = jnp.exp(d * d * cf)
                return _
            lax.fori_loop(0, CHUNK // 16, gbody, 0)
            for cp in cps:
                cp.wait()

            def sbody(e, _):
                ge = gv[e]
                rows[e, pl.ds(0, 16)] = rows[e, pl.ds(0, 16)] * ge
                rows[e, pl.ds(16, 16)] = rows[e, pl.ds(16, 16)] * ge
                return _
            lax.fori_loop(0, CHUNK, sbody, 0)

            for j in range(NSUBC):
                pltpu.sync_copy(rows.at[pl.ds(j * SUB, SUB)],
                                acc.at[dstv.at[j]], add=True)
            return carry

        lax.fori_loop(0, N_CHUNKS, chunk, 0)
        plsc.subcore_barrier()
        pltpu.sync_copy(acc.at[pl.ds(s * ROWS_OUT, ROWS_OUT)],
                        out.at[c, pl.ds(s * ROWS_OUT, ROWS_OUT)])

    return pl.kernel(
        body,
        out_type=jax.ShapeDtypeStruct((2, N, width), jnp.float32),
        mesh=mesh,
        scratch_types=[
            pltpu.VMEM_SHARED((ACC_ROWS, width), jnp.float32),
            pltpu.VMEM((NSUBC, SUB), jnp.int32),
            pltpu.VMEM((NSUBC, SUB), jnp.int32),
            pltpu.VMEM((CHUNK,), jnp.float32),
            pltpu.VMEM((CHUNK,), jnp.float32),
            pltpu.VMEM((CHUNK, width), jnp.float32),
            pltpu.VMEM((16,), jnp.float32),
            pltpu.SemaphoreType.DMA,
        ],
    )


def _full(shape):
    return pl.BlockSpec(shape, lambda i: (0,) * len(shape))


def _tc1_body(x, W1, b1, gW0, rW0, h_o, xa_o, hr_o):
    h = jnp.maximum(
        jnp.dot(x[...], W1[...], preferred_element_type=jnp.float32) + b1[...],
        0.0)
    h_o[...] = h
    xg = jnp.dot(h, gW0[...], preferred_element_type=jnp.float32)
    extra = (lax.broadcasted_iota(jnp.int32, (RB, 16), 1) == 0).astype(jnp.float32)
    xa_o[...] = jnp.concatenate([xg, extra], axis=1)
    hr_o[...] = jnp.dot(h, rW0[...], preferred_element_type=jnp.float32)


def _tc2_body(p1, h, hr1, cb0, f0, gW1, rW1, h2_o, xa2_o, hr2_o, deg_o):
    p = p1[...]
    su = p[0] + p[1]
    deg = jnp.maximum(su[:, 32:33], 1.0)
    h2 = su[:, :32] / deg + hr1[...] + cb0[...] + f0[0, 0] * h[...]
    h2_o[...] = h2
    xa2_o[...] = jnp.dot(h2, gW1[...], preferred_element_type=jnp.float32)
    hr2_o[...] = jnp.dot(h2, rW1[...], preferred_element_type=jnp.float32)
    deg_o[...] = jnp.broadcast_to(deg, (RB, 8))


def _tc3_body(p2, deg8, h2, hr2, cb1, f1, WoP, boP, out_o):
    p = p2[...]
    su = p[0] + p[1]
    deg = deg8[...][:, 0:1]
    h3 = su / deg + hr2[...] + cb1[...] + f1[0, 0] * h2[...]
    lg = jnp.dot(h3, WoP[...], preferred_element_type=jnp.float32) + boP[...]
    mask = lax.broadcasted_iota(jnp.int32, (RB, 128), 1) < C
    m = jnp.max(jnp.where(mask, lg, -1e30), axis=1, keepdims=True)
    ex = jnp.where(mask, jnp.exp(lg - m), 0.0)
    lse = m + jnp.log(jnp.sum(ex, axis=1, keepdims=True))
    out_o[...] = (lg - lse)[:, :8]


def kernel(x, edge_index, edge_weight, W1, b1, gW, rootW, mu, sigma,
           conv_bias, fuse, Wout, bout):
    f32 = jnp.float32
    src = edge_index[0]
    dst = edge_index[1]
    pad = E_PAD - E
    src_p = jnp.concatenate([src, jnp.zeros((pad,), jnp.int32)])
    dst_p = jnp.concatenate([dst, jnp.full((pad,), N, jnp.int32)])
    ew_p = jnp.concatenate([edge_weight[:, 0], jnp.zeros((pad,), f32)])
    src2 = src_p.reshape(-1, SUB)
    dst2 = dst_p.reshape(-1, SUB)

    prm = []
    for l in range(2):
        cf = -0.5 / (EPS + sigma[l, 0, 0] ** 2)
        prm.append(jnp.concatenate([mu[l, 0, 0][None], cf[None],
                                    jnp.zeros((14,), f32)]))

    b1r = b1.reshape(1, H)
    cb0 = conv_bias[0].reshape(1, H)
    cb1 = conv_bias[1].reshape(1, H)
    f0 = fuse[0].reshape(1, 1)
    f1 = fuse[1].reshape(1, 1)
    WoP = jnp.zeros((H, 128), f32).at[:, :C].set(Wout)
    boP = jnp.zeros((1, 128), f32).at[0, :C].set(bout)

    nf = jax.ShapeDtypeStruct((N, H), f32)
    h, xa1, hr1 = pl.pallas_call(
        _tc1_body,
        grid=(GRID,),
        in_specs=[
            pl.BlockSpec((RB, F_IN), lambda i: (i, 0)),
            _full((F_IN, H)), _full((1, H)), _full((H, H)), _full((H, H)),
        ],
        out_specs=[
            pl.BlockSpec((RB, H), lambda i: (i, 0)),
            pl.BlockSpec((RB, 48), lambda i: (i, 0)),
            pl.BlockSpec((RB, H), lambda i: (i, 0)),
        ],
        out_shape=[nf, jax.ShapeDtypeStruct((N, 48), f32), nf],
    )(x, W1, b1r, gW[0], rootW[0])

    zr48 = jnp.zeros((ZROWS, 48), f32)
    p1 = _sc_segment(48)(xa1, src2, dst2, ew_p, prm[0], zr48)

    h2, xa2, hr2, deg8 = pl.pallas_call(
        _tc2_body,
        grid=(GRID,),
        in_specs=[
            pl.BlockSpec((2, RB, 48), lambda i: (0, i, 0)),
            pl.BlockSpec((RB, H), lambda i: (i, 0)),
            pl.BlockSpec((RB, H), lambda i: (i, 0)),
            _full((1, H)), _full((1, 1)), _full((H, H)), _full((H, H)),
        ],
        out_specs=[
            pl.BlockSpec((RB, H), lambda i: (i, 0)),
            pl.BlockSpec((RB, H), lambda i: (i, 0)),
            pl.BlockSpec((RB, H), lambda i: (i, 0)),
            pl.BlockSpec((RB, 8), lambda i: (i, 0)),
        ],
        out_shape=[nf, nf, nf, jax.ShapeDtypeStruct((N, 8), f32)],
    )(p1, h, hr1, cb0, f0, gW[1], rootW[1])

    zr32 = jnp.zeros((ZROWS, H), f32)
    p2 = _sc_segment(H)(xa2, src2, dst2, ew_p, prm[1], zr32)

    out8 = pl.pallas_call(
        _tc3_body,
        grid=(GRID,),
        in_specs=[
            pl.BlockSpec((2, RB, H), lambda i: (0, i, 0)),
            pl.BlockSpec((RB, 8), lambda i: (i, 0)),
            pl.BlockSpec((RB, H), lambda i: (i, 0)),
            pl.BlockSpec((RB, H), lambda i: (i, 0)),
            _full((1, H)), _full((1, 1)), _full((H, 128)), _full((1, 128)),
        ],
        out_specs=[pl.BlockSpec((RB, 8), lambda i: (i, 0))],
        out_shape=[jax.ShapeDtypeStruct((N, 8), f32)],
    )(p2, deg8, h2, hr2, cb1, f1, WoP, boP)[0]

    return out8[:, :C]


# speedup vs baseline: 7.1428x; 7.1428x over previous
import functools

import jax
import jax.numpy as jnp
from jax import lax
from jax.experimental import pallas as pl
from jax.experimental.pallas import tpu as pltpu
from jax.experimental.pallas import tpu_sc as plsc

N = 10000
E = 320000
F_IN = 128
H = 32
C = 2
EPS = 1e-15

NW = 32
NSUB = 16
CHUNK = 1024
SUB = 128
NSUBC = CHUNK // SUB
EPT = 10240
E_PAD = NW * EPT
N_CHUNKS = EPT // CHUNK
ACC_ROWS = 10240
ZROWS = ACC_ROWS // NSUB

RB = 1000
GRID = N // RB


def _sc_segment(width):
    mesh = plsc.VectorSubcoreMesh(core_axis_name="c", subcore_axis_name="s")

    def body(table, src2, dst2, ewr, prm, zrows, out,
             acc, srcv, dstv, ewv, gv, rows, prmv, sem):
        c = lax.axis_index("c")
        s = lax.axis_index("s")
        wid = c * NSUB + s
        pltpu.sync_copy(zrows, acc.at[pl.ds(s * ZROWS, ZROWS)])
        pltpu.sync_copy(prm, prmv)
        plsc.subcore_barrier()
        pv = prmv[...]
        mu = pv[0]
        cf = pv[1]

        def chunk(k, carry):
            base = wid * EPT + k * CHUNK
            rbase = wid * (EPT // SUB) + k * NSUBC
            pltpu.sync_copy(src2.at[pl.ds(rbase, NSUBC)], srcv)
            pltpu.sync_copy(dst2.at[pl.ds(rbase, NSUBC)], dstv)
            pltpu.sync_copy(ewr.at[pl.ds(base, CHUNK)], ewv)
            cps = [
                pltpu.async_copy(table.at[srcv.at[j]],
                                 rows.at[pl.ds(j * SUB, SUB)], sem)
                for j in range(NSUBC)
            ]

            def gbody(j, _):
                w = ewv[pl.ds(j * 16, 16)]
                d = w - mu
                gv[pl.ds(j * 16, 16)] = jnp.exp(d * d * cf)
                return _
            lax.fori_loop(0, CHUNK // 16, gbody, 0)
            for cp in cps:
                cp.wait()

            def sbody(b, _):
                g16 = gv[pl.ds(b * 16, 16)]
                e0 = b * 16
                for j in range(16):
                    ge = g16[j]
                    rows[e0 + j, pl.ds(0, 16)] = rows[e0 + j, pl.ds(0, 16)] * ge
                    rows[e0 + j, pl.ds(16, 16)] = rows[e0 + j, pl.ds(16, 16)] * ge
                return _
            lax.fori_loop(0, CHUNK // 16, sbody, 0)

            for j in range(NSUBC):
                pltpu.sync_copy(rows.at[pl.ds(j * SUB, SUB)],
                                acc.at[dstv.at[j]], add=True)
            return carry

        lax.fori_loop(0, N_CHUNKS, chunk, 0)
        plsc.subcore_barrier()
        pltpu.sync_copy(acc.at[pl.ds(s * ZROWS, ZROWS)],
                        out.at[c, pl.ds(s * ZROWS, ZROWS)])

    return pl.kernel(
        body,
        out_type=jax.ShapeDtypeStruct((2, ACC_ROWS, width), jnp.float32),
        mesh=mesh,
        compiler_params=pltpu.CompilerParams(use_tc_tiling_on_sc=False),
        scratch_types=[
            pltpu.VMEM_SHARED((ACC_ROWS, width), jnp.float32),
            pltpu.VMEM((NSUBC, SUB), jnp.int32),
            pltpu.VMEM((NSUBC, SUB), jnp.int32),
            pltpu.VMEM((CHUNK,), jnp.float32),
            pltpu.VMEM((CHUNK,), jnp.float32),
            pltpu.VMEM((CHUNK, width), jnp.float32),
            pltpu.VMEM((16,), jnp.float32),
            pltpu.SemaphoreType.DMA,
        ],
    )


def _full(shape):
    return pl.BlockSpec(shape, lambda i: (0,) * len(shape))


def _tc1_body(x, W1, b1, gW0, rW0, h_o, xa_o, hr_o):
    h = jnp.maximum(
        jnp.dot(x[...], W1[...], preferred_element_type=jnp.float32) + b1[...],
        0.0)
    h_o[...] = h
    xg = jnp.dot(h, gW0[...], preferred_element_type=jnp.float32)
    extra = (lax.broadcasted_iota(jnp.int32, (RB, 16), 1) == 0).astype(jnp.float32)
    xa_o[...] = jnp.concatenate([xg, extra], axis=1)
    hr_o[...] = jnp.dot(h, rW0[...], preferred_element_type=jnp.float32)


def _tc2_body(p1, h, hr1, cb0, f0, gW1, rW1, h2_o, xa2_o, hr2_o, deg_o):
    p = p1[...]
    su = p[0] + p[1]
    deg = jnp.maximum(su[:, 32:33], 1.0)
    h2 = su[:, :32] / deg + hr1[...] + cb0[...] + f0[0, 0] * h[...]
    h2_o[...] = h2
    xa2_o[...] = jnp.dot(h2, gW1[...], preferred_element_type=jnp.float32)
    hr2_o[...] = jnp.dot(h2, rW1[...], preferred_element_type=jnp.float32)
    deg_o[...] = jnp.broadcast_to(deg, (RB, 8))


def _tc3_body(p2, deg8, h2, hr2, cb1, f1, WoP, boP, out_o):
    p = p2[...]
    su = p[0] + p[1]
    deg = deg8[...][:, 0:1]
    h3 = su / deg + hr2[...] + cb1[...] + f1[0, 0] * h2[...]
    lg = jnp.dot(h3, WoP[...], preferred_element_type=jnp.float32) + boP[...]
    mask = lax.broadcasted_iota(jnp.int32, (RB, 128), 1) < C
    m = jnp.max(jnp.where(mask, lg, -1e30), axis=1, keepdims=True)
    ex = jnp.where(mask, jnp.exp(lg - m), 0.0)
    lse = m + jnp.log(jnp.sum(ex, axis=1, keepdims=True))
    out_o[...] = (lg - lse)[:, :8]


def kernel(x, edge_index, edge_weight, W1, b1, gW, rootW, mu, sigma,
           conv_bias, fuse, Wout, bout):
    f32 = jnp.float32
    src = edge_index[0]
    dst = edge_index[1]
    pad = E_PAD - E
    src_p = jnp.concatenate([src, jnp.zeros((pad,), jnp.int32)])
    dst_p = jnp.concatenate([dst, jnp.full((pad,), N, jnp.int32)])
    ew_p = jnp.concatenate([edge_weight[:, 0], jnp.zeros((pad,), f32)])
    src2 = src_p.reshape(-1, SUB)
    dst2 = dst_p.reshape(-1, SUB)

    prm = []
    for l in range(2):
        cf = -0.5 / (EPS + sigma[l, 0, 0] ** 2)
        prm.append(jnp.concatenate([mu[l, 0, 0][None], cf[None],
                                    jnp.zeros((14,), f32)]))

    b1r = b1.reshape(1, H)
    cb0 = conv_bias[0].reshape(1, H)
    cb1 = conv_bias[1].reshape(1, H)
    f0 = fuse[0].reshape(1, 1)
    f1 = fuse[1].reshape(1, 1)
    WoP = jnp.zeros((H, 128), f32).at[:, :C].set(Wout)
    boP = jnp.zeros((1, 128), f32).at[0, :C].set(bout)

    nf = jax.ShapeDtypeStruct((N, H), f32)
    h, xa1, hr1 = pl.pallas_call(
        _tc1_body,
        grid=(GRID,),
        in_specs=[
            pl.BlockSpec((RB, F_IN), lambda i: (i, 0)),
            _full((F_IN, H)), _full((1, H)), _full((H, H)), _full((H, H)),
        ],
        out_specs=[
            pl.BlockSpec((RB, H), lambda i: (i, 0)),
            pl.BlockSpec((RB, 48), lambda i: (i, 0)),
            pl.BlockSpec((RB, H), lambda i: (i, 0)),
        ],
        out_shape=[nf, jax.ShapeDtypeStruct((N, 48), f32), nf],
    )(x, W1, b1r, gW[0], rootW[0])

    zr48 = jnp.zeros((ZROWS, 48), f32)
    p1 = _sc_segment(48)(xa1, src2, dst2, ew_p, prm[0], zr48)

    h2, xa2, hr2, deg8 = pl.pallas_call(
        _tc2_body,
        grid=(GRID,),
        in_specs=[
            pl.BlockSpec((2, RB, 48), lambda i: (0, i, 0)),
            pl.BlockSpec((RB, H), lambda i: (i, 0)),
            pl.BlockSpec((RB, H), lambda i: (i, 0)),
            _full((1, H)), _full((1, 1)), _full((H, H)), _full((H, H)),
        ],
        out_specs=[
            pl.BlockSpec((RB, H), lambda i: (i, 0)),
            pl.BlockSpec((RB, H), lambda i: (i, 0)),
            pl.BlockSpec((RB, H), lambda i: (i, 0)),
            pl.BlockSpec((RB, 8), lambda i: (i, 0)),
        ],
        out_shape=[nf, nf, nf, jax.ShapeDtypeStruct((N, 8), f32)],
    )(p1, h, hr1, cb0, f0, gW[1], rootW[1])

    zr32 = jnp.zeros((ZROWS, H), f32)
    p2 = _sc_segment(H)(xa2, src2, dst2, ew_p, prm[1], zr32)

    out8 = pl.pallas_call(
        _tc3_body,
        grid=(GRID,),
        in_specs=[
            pl.BlockSpec((2, RB, H), lambda i: (0, i, 0)),
            pl.BlockSpec((RB, 8), lambda i: (i, 0)),
            pl.BlockSpec((RB, H), lambda i: (i, 0)),
            pl.BlockSpec((RB, H), lambda i: (i, 0)),
            _full((1, H)), _full((1, 1)), _full((H, 128)), _full((1, 128)),
        ],
        out_specs=[pl.BlockSpec((RB, 8), lambda i: (i, 0))],
        out_shape=[jax.ShapeDtypeStruct((N, 8), f32)],
    )(p2, deg8, h2, hr2, cb1, f1, WoP, boP)[0]

    return out8[:, :C]


# speedup vs baseline: 8.4214x; 1.1790x over previous
import functools

import jax
import jax.numpy as jnp
from jax import lax
from jax.experimental import pallas as pl
from jax.experimental.pallas import tpu as pltpu
from jax.experimental.pallas import tpu_sc as plsc

N = 10000
E = 320000
F_IN = 128
H = 32
C = 2
EPS = 1e-15

NW = 32
NSUB = 16
CHUNK = 1024
SUB = 128
NSUBC = CHUNK // SUB
EPT = 10240
E_PAD = NW * EPT
N_CHUNKS = EPT // CHUNK
ACC_ROWS = 10240
ZROWS = ACC_ROWS // NSUB

RB = 1000
GRID = N // RB


RPT = EPT // SUB


def _sc_segment(width, chunk):
    nsubc = chunk // SUB
    n_chunks = EPT // chunk
    mesh = plsc.VectorSubcoreMesh(core_axis_name="c", subcore_axis_name="s")

    def body(table, ipk, ewr, prm, zrows, out,
             acc, ipack, ewv, gv, rows, prmv, gs0, gs1, ss0, ss1):
        c = lax.axis_index("c")
        s = lax.axis_index("s")
        wid = c * NSUB + s
        gsem = [gs0, gs1]
        ssem = [ss0, ss1]
        pltpu.sync_copy(ipk.at[pl.ds(wid * RPT, RPT)], ipack)
        pltpu.sync_copy(ewr.at[pl.ds(wid * EPT, EPT)], ewv)

        descs_g = [None, None]
        descs_s = [None, None]

        def fire_gathers(k):
            b = k % 2
            descs_g[b] = [
                pltpu.async_copy(table.at[ipack.at[k * nsubc + j, 0]],
                                 rows.at[b, pl.ds(j * SUB, SUB)], gsem[b])
                for j in range(nsubc)
            ]

        fire_gathers(0)
        pltpu.sync_copy(zrows, acc.at[pl.ds(s * ZROWS, ZROWS)])
        pltpu.sync_copy(prm, prmv)
        pv = prmv[...]
        mu = pv[0]
        cf = pv[1]

        def gbody(i, _):
            w = ewv[pl.ds(i * 16, 16)]
            d = w - mu
            gv[pl.ds(i * 16, 16)] = jnp.exp(d * d * cf)
            return _
        lax.fori_loop(0, EPT // 16, gbody, 0)
        plsc.subcore_barrier()

        for k in range(n_chunks):
            b = k % 2
            if k + 1 < n_chunks:
                b1 = (k + 1) % 2
                if descs_s[b1] is not None:
                    for dd in descs_s[b1]:
                        dd.wait()
                fire_gathers(k + 1)
            for dd in descs_g[b]:
                dd.wait()

            def sbody(bb, _):
                g16 = gv[pl.ds(k * chunk + bb * 16, 16)]
                e0 = bb * 16
                for j in range(16):
                    ge = g16[j]
                    rows[b, e0 + j, pl.ds(0, 16)] = (
                        rows[b, e0 + j, pl.ds(0, 16)] * ge)
                    rows[b, e0 + j, pl.ds(16, 16)] = (
                        rows[b, e0 + j, pl.ds(16, 16)] * ge)
                return _
            lax.fori_loop(0, chunk // 16, sbody, 0)

            descs_s[b] = [
                pltpu.async_copy(rows.at[b, pl.ds(j * SUB, SUB)],
                                 acc.at[ipack.at[k * nsubc + j, 1]],
                                 ssem[b], add=True)
                for j in range(nsubc)
            ]
        for b in range(2):
            if descs_s[b] is not None:
                for dd in descs_s[b]:
                    dd.wait()
        plsc.subcore_barrier()
        pltpu.sync_copy(acc.at[pl.ds(s * ZROWS, ZROWS)],
                        out.at[c, pl.ds(s * ZROWS, ZROWS)])

    return pl.kernel(
        body,
        out_type=jax.ShapeDtypeStruct((2, ACC_ROWS, width), jnp.float32),
        mesh=mesh,
        compiler_params=pltpu.CompilerParams(use_tc_tiling_on_sc=False),
        scratch_types=[
            pltpu.VMEM_SHARED((ACC_ROWS, width), jnp.float32),
            pltpu.VMEM((RPT, 2, SUB), jnp.int32),
            pltpu.VMEM((EPT,), jnp.float32),
            pltpu.VMEM((EPT,), jnp.float32),
            pltpu.VMEM((2, chunk, width), jnp.float32),
            pltpu.VMEM((16,), jnp.float32),
            pltpu.SemaphoreType.DMA,
            pltpu.SemaphoreType.DMA,
            pltpu.SemaphoreType.DMA,
            pltpu.SemaphoreType.DMA,
        ],
    )


def _full(shape):
    return pl.BlockSpec(shape, lambda i: (0,) * len(shape))


def _tc1_body(x, W1, b1, gW0, rW0, h_o, xa_o, hr_o):
    h = jnp.maximum(
        jnp.dot(x[...], W1[...], preferred_element_type=jnp.float32) + b1[...],
        0.0)
    h_o[...] = h
    xg = jnp.dot(h, gW0[...], preferred_element_type=jnp.float32)
    extra = (lax.broadcasted_iota(jnp.int32, (RB, 16), 1) == 0).astype(jnp.float32)
    xa_o[...] = jnp.concatenate([xg, extra], axis=1)
    hr_o[...] = jnp.dot(h, rW0[...], preferred_element_type=jnp.float32)


def _tc2_body(p1, h, hr1, cb0, f0, gW1, rW1, h2_o, xa2_o, hr2_o, deg_o):
    p = p1[...]
    su = p[0] + p[1]
    deg = jnp.maximum(su[:, 32:33], 1.0)
    h2 = su[:, :32] / deg + hr1[...] + cb0[...] + f0[0, 0] * h[...]
    h2_o[...] = h2
    xa2_o[...] = jnp.dot(h2, gW1[...], preferred_element_type=jnp.float32)
    hr2_o[...] = jnp.dot(h2, rW1[...], preferred_element_type=jnp.float32)
    deg_o[...] = jnp.broadcast_to(deg, (RB, 8))


def _tc3_body(p2, deg8, h2, hr2, cb1, f1, WoP, boP, out_o):
    p = p2[...]
    su = p[0] + p[1]
    deg = deg8[...][:, 0:1]
    h3 = su / deg + hr2[...] + cb1[...] + f1[0, 0] * h2[...]
    lg = jnp.dot(h3, WoP[...], preferred_element_type=jnp.float32) + boP[...]
    mask = lax.broadcasted_iota(jnp.int32, (RB, 128), 1) < C
    m = jnp.max(jnp.where(mask, lg, -1e30), axis=1, keepdims=True)
    ex = jnp.where(mask, jnp.exp(lg - m), 0.0)
    lse = m + jnp.log(jnp.sum(ex, axis=1, keepdims=True))
    out_o[...] = (lg - lse)[:, :8]


def kernel(x, edge_index, edge_weight, W1, b1, gW, rootW, mu, sigma,
           conv_bias, fuse, Wout, bout):
    f32 = jnp.float32
    src = edge_index[0]
    dst = edge_index[1]
    pad = E_PAD - E
    src_p = jnp.concatenate([src, jnp.zeros((pad,), jnp.int32)])
    dst_p = jnp.concatenate([dst, jnp.full((pad,), N, jnp.int32)])
    ew_p = jnp.concatenate([edge_weight[:, 0], jnp.zeros((pad,), f32)])
    ipk = jnp.stack([
        src_p.reshape(-1, SUB),
        dst_p.reshape(-1, SUB),
    ], axis=1)

    prm = []
    for l in range(2):
        cf = -0.5 / (EPS + sigma[l, 0, 0] ** 2)
        prm.append(jnp.concatenate([mu[l, 0, 0][None], cf[None],
                                    jnp.zeros((14,), f32)]))

    b1r = b1.reshape(1, H)
    cb0 = conv_bias[0].reshape(1, H)
    cb1 = conv_bias[1].reshape(1, H)
    f0 = fuse[0].reshape(1, 1)
    f1 = fuse[1].reshape(1, 1)
    WoP = jnp.zeros((H, 128), f32).at[:, :C].set(Wout)
    boP = jnp.zeros((1, 128), f32).at[0, :C].set(bout)

    nf = jax.ShapeDtypeStruct((N, H), f32)
    h, xa1, hr1 = pl.pallas_call(
        _tc1_body,
        grid=(GRID,),
        in_specs=[
            pl.BlockSpec((RB, F_IN), lambda i: (i, 0)),
            _full((F_IN, H)), _full((1, H)), _full((H, H)), _full((H, H)),
        ],
        out_specs=[
            pl.BlockSpec((RB, H), lambda i: (i, 0)),
            pl.BlockSpec((RB, 48), lambda i: (i, 0)),
            pl.BlockSpec((RB, H), lambda i: (i, 0)),
        ],
        out_shape=[nf, jax.ShapeDtypeStruct((N, 48), f32), nf],
    )(x, W1, b1r, gW[0], rootW[0])

    zr48 = jnp.zeros((ZROWS, 48), f32)
    p1 = _sc_segment(48, 512)(xa1, ipk, ew_p, prm[0], zr48)

    h2, xa2, hr2, deg8 = pl.pallas_call(
        _tc2_body,
        grid=(GRID,),
        in_specs=[
            pl.BlockSpec((2, RB, 48), lambda i: (0, i, 0)),
            pl.BlockSpec((RB, H), lambda i: (i, 0)),
            pl.BlockSpec((RB, H), lambda i: (i, 0)),
            _full((1, H)), _full((1, 1)), _full((H, H)), _full((H, H)),
        ],
        out_specs=[
            pl.BlockSpec((RB, H), lambda i: (i, 0)),
            pl.BlockSpec((RB, H), lambda i: (i, 0)),
            pl.BlockSpec((RB, H), lambda i: (i, 0)),
            pl.BlockSpec((RB, 8), lambda i: (i, 0)),
        ],
        out_shape=[nf, nf, nf, jax.ShapeDtypeStruct((N, 8), f32)],
    )(p1, h, hr1, cb0, f0, gW[1], rootW[1])

    zr32 = jnp.zeros((ZROWS, H), f32)
    p2 = _sc_segment(H, 1024)(xa2, ipk, ew_p, prm[1], zr32)

    out8 = pl.pallas_call(
        _tc3_body,
        grid=(GRID,),
        in_specs=[
            pl.BlockSpec((2, RB, H), lambda i: (0, i, 0)),
            pl.BlockSpec((RB, 8), lambda i: (i, 0)),
            pl.BlockSpec((RB, H), lambda i: (i, 0)),
            pl.BlockSpec((RB, H), lambda i: (i, 0)),
            _full((1, H)), _full((1, 1)), _full((H, 128)), _full((1, 128)),
        ],
        out_specs=[pl.BlockSpec((RB, 8), lambda i: (i, 0))],
        out_shape=[jax.ShapeDtypeStruct((N, 8), f32)],
    )(p2, deg8, h2, hr2, cb1, f1, WoP, boP)[0]

    return out8[:, :C]


# speedup vs baseline: 15.2747x; 1.8138x over previous
import functools

import jax
import jax.numpy as jnp
from jax import lax
from jax.experimental import pallas as pl
from jax.experimental.pallas import tpu as pltpu
from jax.experimental.pallas import tpu_sc as plsc

N = 10000
E = 320000
F_IN = 128
H = 32
C = 2
EPS = 1e-15

NW = 32
NSUB = 16
CHUNK = 1024
SUB = 128
NSUBC = CHUNK // SUB
EPT = 10240
E_PAD = NW * EPT
N_CHUNKS = EPT // CHUNK
ACC_ROWS = 10240
ZROWS = ACC_ROWS // NSUB

RB = 1000
GRID = N // RB


RPT = EPT // SUB


def _sc_segment(width, chunk):
    nsubc = chunk // SUB
    n_chunks = EPT // chunk
    mesh = plsc.VectorSubcoreMesh(core_axis_name="c", subcore_axis_name="s")

    def body(table, ipk, ewr, prm, zrows, out,
             acc, ipack, ewv, gv, rows, prmv, gs0, gs1, ss0, ss1):
        c = lax.axis_index("c")
        s = lax.axis_index("s")
        wid = c * NSUB + s
        gsem = [gs0, gs1]
        ssem = [ss0, ss1]
        pltpu.sync_copy(ipk.at[pl.ds(wid * RPT, RPT)], ipack)
        pltpu.sync_copy(ewr.at[pl.ds(wid * EPT, EPT)], ewv)

        descs_g = [None, None]
        descs_s = [None, None]

        def fire_gathers(k):
            b = k % 2
            descs_g[b] = [
                pltpu.async_copy(table.at[ipack.at[k * nsubc + j, 0]],
                                 rows.at[b, pl.ds(j * SUB, SUB)], gsem[b])
                for j in range(nsubc)
            ]

        fire_gathers(0)
        pltpu.sync_copy(zrows, acc.at[pl.ds(s * ZROWS, ZROWS)])
        pltpu.sync_copy(prm, prmv)
        pv = prmv[...]
        mu = pv[0]
        cf = pv[1]

        def gbody(i, _):
            w = ewv[pl.ds(i * 16, 16)]
            d = w - mu
            gv[pl.ds(i * 16, 16)] = jnp.exp(d * d * cf)
            return _
        lax.fori_loop(0, EPT // 16, gbody, 0)
        plsc.subcore_barrier()

        for k in range(n_chunks):
            b = k % 2
            if k + 1 < n_chunks:
                b1 = (k + 1) % 2
                if descs_s[b1] is not None:
                    for dd in descs_s[b1]:
                        dd.wait()
                fire_gathers(k + 1)
            for dd in descs_g[b]:
                dd.wait()

            def sbody(bb, _):
                g16 = gv[pl.ds(k * chunk + bb * 16, 16)]
                e0 = bb * 16
                for j in range(16):
                    ge = g16[j]
                    rows[b, e0 + j, pl.ds(0, 16)] = (
                        rows[b, e0 + j, pl.ds(0, 16)] * ge)
                    rows[b, e0 + j, pl.ds(16, 16)] = (
                        rows[b, e0 + j, pl.ds(16, 16)] * ge)
                return _
            lax.fori_loop(0, chunk // 16, sbody, 0)

            descs_s[b] = [
                pltpu.async_copy(rows.at[b, pl.ds(j * SUB, SUB)],
                                 acc.at[ipack.at[k * nsubc + j, 1]],
                                 ssem[b], add=True)
                for j in range(nsubc)
            ]
        for b in range(2):
            if descs_s[b] is not None:
                for dd in descs_s[b]:
                    dd.wait()
        plsc.subcore_barrier()
        pltpu.sync_copy(acc.at[pl.ds(s * ZROWS, ZROWS)],
                        out.at[c, pl.ds(s * ZROWS, ZROWS)])

    return pl.kernel(
        body,
        out_type=jax.ShapeDtypeStruct((2, ACC_ROWS, width), jnp.float32),
        mesh=mesh,
        compiler_params=pltpu.CompilerParams(use_tc_tiling_on_sc=False),
        scratch_types=[
            pltpu.VMEM_SHARED((ACC_ROWS, width), jnp.float32),
            pltpu.VMEM((RPT, 2, SUB), jnp.int32),
            pltpu.VMEM((EPT,), jnp.float32),
            pltpu.VMEM((EPT,), jnp.float32),
            pltpu.VMEM((2, chunk, width), jnp.float32),
            pltpu.VMEM((16,), jnp.float32),
            pltpu.SemaphoreType.DMA,
            pltpu.SemaphoreType.DMA,
            pltpu.SemaphoreType.DMA,
            pltpu.SemaphoreType.DMA,
        ],
    )


def _full(shape):
    return pl.BlockSpec(shape, lambda i: (0,) * len(shape))


def _tc1_body(x, W1, b1, gW0, rW0, h_o, xa_o, hr_o):
    h = jnp.maximum(
        jnp.dot(x[...], W1[...], preferred_element_type=jnp.float32) + b1[...],
        0.0)
    h_o[...] = h
    xg = jnp.dot(h, gW0[...], preferred_element_type=jnp.float32)
    extra = (lax.broadcasted_iota(jnp.int32, (RB, 16), 1) == 0).astype(jnp.float32)
    xa_o[...] = jnp.concatenate([xg, extra], axis=1)
    hr_o[...] = jnp.dot(h, rW0[...], preferred_element_type=jnp.float32)


def _tc2_body(p1, h, hr1, cb0, f0, gW1, rW1, h2_o, xa2_o, hr2_o, deg_o):
    p = p1[...]
    su = p[0] + p[1]
    deg = jnp.maximum(su[:, 32:33], 1.0)
    h2 = su[:, :32] / deg + hr1[...] + cb0[...] + f0[0, 0] * h[...]
    h2_o[...] = h2
    xa2_o[...] = jnp.dot(h2, gW1[...], preferred_element_type=jnp.float32)
    hr2_o[...] = jnp.dot(h2, rW1[...], preferred_element_type=jnp.float32)
    deg_o[...] = jnp.broadcast_to(deg, (RB, 8))


def _tc3_body(p2, deg8, h2, hr2, cb1, f1, WoP, boP, out_o):
    p = p2[...]
    su = p[0] + p[1]
    deg = deg8[...][:, 0:1]
    h3 = su / deg + hr2[...] + cb1[...] + f1[0, 0] * h2[...]
    lg = jnp.dot(h3, WoP[...], preferred_element_type=jnp.float32) + boP[...]
    mask = lax.broadcasted_iota(jnp.int32, (RB, 128), 1) < C
    m = jnp.max(jnp.where(mask, lg, -1e30), axis=1, keepdims=True)
    ex = jnp.where(mask, jnp.exp(lg - m), 0.0)
    lse = m + jnp.log(jnp.sum(ex, axis=1, keepdims=True))
    out_o[...] = (lg - lse)[:, :8]


def kernel(x, edge_index, edge_weight, W1, b1, gW, rootW, mu, sigma,
           conv_bias, fuse, Wout, bout):
    f32 = jnp.float32
    src = edge_index[0]
    dst = edge_index[1]
    pad = E_PAD - E
    pad_i = jnp.arange(pad, dtype=jnp.int32)
    src_p = jnp.concatenate([src, pad_i % N])
    dst_p = jnp.concatenate([dst, N + pad_i % (ACC_ROWS - N)])
    ew_p = jnp.concatenate([edge_weight[:, 0], jnp.zeros((pad,), f32)])
    ipk = jnp.stack([
        src_p.reshape(-1, SUB),
        dst_p.reshape(-1, SUB),
    ], axis=1)

    prm = []
    for l in range(2):
        cf = -0.5 / (EPS + sigma[l, 0, 0] ** 2)
        prm.append(jnp.concatenate([mu[l, 0, 0][None], cf[None],
                                    jnp.zeros((14,), f32)]))

    b1r = b1.reshape(1, H)
    cb0 = conv_bias[0].reshape(1, H)
    cb1 = conv_bias[1].reshape(1, H)
    f0 = fuse[0].reshape(1, 1)
    f1 = fuse[1].reshape(1, 1)
    WoP = jnp.zeros((H, 128), f32).at[:, :C].set(Wout)
    boP = jnp.zeros((1, 128), f32).at[0, :C].set(bout)

    nf = jax.ShapeDtypeStruct((N, H), f32)
    h, xa1, hr1 = pl.pallas_call(
        _tc1_body,
        grid=(GRID,),
        in_specs=[
            pl.BlockSpec((RB, F_IN), lambda i: (i, 0)),
            _full((F_IN, H)), _full((1, H)), _full((H, H)), _full((H, H)),
        ],
        out_specs=[
            pl.BlockSpec((RB, H), lambda i: (i, 0)),
            pl.BlockSpec((RB, 48), lambda i: (i, 0)),
            pl.BlockSpec((RB, H), lambda i: (i, 0)),
        ],
        out_shape=[nf, jax.ShapeDtypeStruct((N, 48), f32), nf],
    )(x, W1, b1r, gW[0], rootW[0])

    zr48 = jnp.zeros((ZROWS, 48), f32)
    p1 = _sc_segment(48, 512)(xa1, ipk, ew_p, prm[0], zr48)

    h2, xa2, hr2, deg8 = pl.pallas_call(
        _tc2_body,
        grid=(GRID,),
        in_specs=[
            pl.BlockSpec((2, RB, 48), lambda i: (0, i, 0)),
            pl.BlockSpec((RB, H), lambda i: (i, 0)),
            pl.BlockSpec((RB, H), lambda i: (i, 0)),
            _full((1, H)), _full((1, 1)), _full((H, H)), _full((H, H)),
        ],
        out_specs=[
            pl.BlockSpec((RB, H), lambda i: (i, 0)),
            pl.BlockSpec((RB, H), lambda i: (i, 0)),
            pl.BlockSpec((RB, H), lambda i: (i, 0)),
            pl.BlockSpec((RB, 8), lambda i: (i, 0)),
        ],
        out_shape=[nf, nf, nf, jax.ShapeDtypeStruct((N, 8), f32)],
    )(p1, h, hr1, cb0, f0, gW[1], rootW[1])

    zr32 = jnp.zeros((ZROWS, H), f32)
    p2 = _sc_segment(H, 1024)(xa2, ipk, ew_p, prm[1], zr32)

    out8 = pl.pallas_call(
        _tc3_body,
        grid=(GRID,),
        in_specs=[
            pl.BlockSpec((2, RB, H), lambda i: (0, i, 0)),
            pl.BlockSpec((RB, 8), lambda i: (i, 0)),
            pl.BlockSpec((RB, H), lambda i: (i, 0)),
            pl.BlockSpec((RB, H), lambda i: (i, 0)),
            _full((1, H)), _full((1, 1)), _full((H, 128)), _full((1, 128)),
        ],
        out_specs=[pl.BlockSpec((RB, 8), lambda i: (i, 0))],
        out_shape=[jax.ShapeDtypeStruct((N, 8), f32)],
    )(p2, deg8, h2, hr2, cb1, f1, WoP, boP)[0]

    return out8[:, :C]


# speedup vs baseline: 15.5538x; 1.0183x over previous
import functools

import jax
import jax.numpy as jnp
from jax import lax
from jax.experimental import pallas as pl
from jax.experimental.pallas import tpu as pltpu
from jax.experimental.pallas import tpu_sc as plsc

N = 10000
E = 320000
F_IN = 128
H = 32
C = 2
EPS = 1e-15

NW = 32
NSUB = 16
CHUNK = 1024
SUB = 128
NSUBC = CHUNK // SUB
EPT = 10240
E_PAD = NW * EPT
N_CHUNKS = EPT // CHUNK
ACC_ROWS = 10240
ZROWS = ACC_ROWS // NSUB

RB = 2000
GRID = N // RB

EROWS = E // SUB
PROWS = E_PAD // SUB
PB = 20
PGRID = PROWS // PB
PREAL = EROWS // PB


RPT = EPT // SUB


def _sc_segment(width, chunk):
    nsubc = chunk // SUB
    n_chunks = EPT // chunk
    mesh = plsc.VectorSubcoreMesh(core_axis_name="c", subcore_axis_name="s")

    def body(table, ipk, ew2, prm, zrows, out,
             acc, ipack, ewv, gv, rows, prmv, gs0, gs1, ss0, ss1):
        c = lax.axis_index("c")
        s = lax.axis_index("s")
        wid = c * NSUB + s
        gsem = [gs0, gs1]
        ssem = [ss0, ss1]
        pltpu.sync_copy(ipk.at[0, pl.ds(wid * RPT, RPT)], ipack.at[0])
        pltpu.sync_copy(ipk.at[1, pl.ds(wid * RPT, RPT)], ipack.at[1])
        pltpu.sync_copy(ew2.at[pl.ds(wid * RPT, RPT)], ewv)

        descs_g = [None, None]
        descs_s = [None, None]

        def fire_gathers(k):
            b = k % 2
            descs_g[b] = [
                pltpu.async_copy(table.at[ipack.at[0, k * nsubc + j]],
                                 rows.at[b, pl.ds(j * SUB, SUB)], gsem[b])
                for j in range(nsubc)
            ]

        fire_gathers(0)
        pltpu.sync_copy(zrows, acc.at[pl.ds(s * ZROWS, ZROWS)])
        pltpu.sync_copy(prm, prmv)
        pv = prmv[...]
        mu = pv[0]
        cf = pv[1]

        def gbody(i, _):
            w = ewv[i >> 3, pl.ds((i & 7) * 16, 16)]
            d = w - mu
            gv[pl.ds(i * 16, 16)] = jnp.exp(d * d * cf)
            return _
        lax.fori_loop(0, EPT // 16, gbody, 0)
        plsc.subcore_barrier()

        for k in range(n_chunks):
            b = k % 2
            if k + 1 < n_chunks:
                b1 = (k + 1) % 2
                if descs_s[b1] is not None:
                    for dd in descs_s[b1]:
                        dd.wait()
                fire_gathers(k + 1)
            for dd in descs_g[b]:
                dd.wait()

            def sbody(bb, _):
                g16 = gv[pl.ds(k * chunk + bb * 16, 16)]
                e0 = bb * 16
                for j in range(16):
                    ge = g16[j]
                    rows[b, e0 + j, pl.ds(0, 16)] = (
                        rows[b, e0 + j, pl.ds(0, 16)] * ge)
                    rows[b, e0 + j, pl.ds(16, 16)] = (
                        rows[b, e0 + j, pl.ds(16, 16)] * ge)
                return _
            lax.fori_loop(0, chunk // 16, sbody, 0)

            descs_s[b] = [
                pltpu.async_copy(rows.at[b, pl.ds(j * SUB, SUB)],
                                 acc.at[ipack.at[1, k * nsubc + j]],
                                 ssem[b], add=True)
                for j in range(nsubc)
            ]
        for b in range(2):
            if descs_s[b] is not None:
                for dd in descs_s[b]:
                    dd.wait()
        plsc.subcore_barrier()
        pltpu.sync_copy(acc.at[pl.ds(s * ZROWS, ZROWS)],
                        out.at[c, pl.ds(s * ZROWS, ZROWS)])

    return pl.kernel(
        body,
        out_type=jax.ShapeDtypeStruct((2, ACC_ROWS, width), jnp.float32),
        mesh=mesh,
        compiler_params=pltpu.CompilerParams(use_tc_tiling_on_sc=False),
        scratch_types=[
            pltpu.VMEM_SHARED((ACC_ROWS, width), jnp.float32),
            pltpu.VMEM((2, RPT, SUB), jnp.int32),
            pltpu.VMEM((RPT, SUB), jnp.float32),
            pltpu.VMEM((EPT,), jnp.float32),
            pltpu.VMEM((2, chunk, width), jnp.float32),
            pltpu.VMEM((16,), jnp.float32),
            pltpu.SemaphoreType.DMA,
            pltpu.SemaphoreType.DMA,
            pltpu.SemaphoreType.DMA,
            pltpu.SemaphoreType.DMA,
        ],
    )


def _full(shape):
    return pl.BlockSpec(shape, lambda i: (0,) * len(shape))


def _prep_body(ei, ew, ipk_o, ew_o):
    npad = PROWS - EROWS
    row_i = lax.broadcasted_iota(jnp.int32, (npad, SUB), 0)
    lane_i = lax.broadcasted_iota(jnp.int32, (npad, SUB), 1)
    pad_i = row_i * SUB + lane_i
    srcp = pad_i % N
    dstp = N + pad_i % (ACC_ROWS - N)
    real = ei[...].reshape(2, EROWS, SUB)
    pads = jnp.stack([srcp, dstp], axis=0)
    ipk_o[...] = jnp.concatenate([real, pads], axis=1)
    ew_o[...] = jnp.concatenate(
        [ew[...], jnp.zeros((npad, SUB), jnp.float32)], axis=0)


def _tc1_body(x, W1, b1, gW0, rW0, h_o, xa_o, hr_o):
    h = jnp.maximum(
        jnp.dot(x[...], W1[...], preferred_element_type=jnp.float32) + b1[...],
        0.0)
    h_o[...] = h
    xg = jnp.dot(h, gW0[...], preferred_element_type=jnp.float32)
    extra = (lax.broadcasted_iota(jnp.int32, (RB, 16), 1) == 0).astype(jnp.float32)
    xa_o[...] = jnp.concatenate([xg, extra], axis=1)
    hr_o[...] = jnp.dot(h, rW0[...], preferred_element_type=jnp.float32)


def _tc2_body(p1, h, hr1, cb0, f0, gW1, rW1, h2_o, xa2_o, hr2_o, deg_o):
    p = p1[...]
    su = p[0] + p[1]
    deg = jnp.maximum(su[:, 32:33], 1.0)
    h2 = su[:, :32] / deg + hr1[...] + cb0[...] + f0[0, 0] * h[...]
    h2_o[...] = h2
    xa2_o[...] = jnp.dot(h2, gW1[...], preferred_element_type=jnp.float32)
    hr2_o[...] = jnp.dot(h2, rW1[...], preferred_element_type=jnp.float32)
    deg_o[...] = jnp.broadcast_to(deg, (RB, 8))


def _tc3_body(p2, deg8, h2, hr2, cb1, f1, Wo, bo, out_o):
    p = p2[...]
    su = p[0] + p[1]
    deg = deg8[...][:, 0:1]
    h3 = su / deg + hr2[...] + cb1[...] + f1[0, 0] * h2[...]
    WoP = jnp.concatenate([Wo[...], jnp.zeros((H, 8 - C), jnp.float32)], 1)
    boP = jnp.concatenate([bo[...], jnp.zeros((1, 8 - C), jnp.float32)], 1)
    lg = jnp.dot(h3, WoP, preferred_element_type=jnp.float32) + boP
    mask = lax.broadcasted_iota(jnp.int32, (RB, 8), 1) < C
    m = jnp.max(jnp.where(mask, lg, -1e30), axis=1, keepdims=True)
    ex = jnp.where(mask, jnp.exp(lg - m), 0.0)
    lse = m + jnp.log(jnp.sum(ex, axis=1, keepdims=True))
    out_o[...] = lg - lse


def kernel(x, edge_index, edge_weight, W1, b1, gW, rootW, mu, sigma,
           conv_bias, fuse, Wout, bout):
    f32 = jnp.float32
    ipk, ew2 = pl.pallas_call(
        _prep_body,
        out_shape=[
            jax.ShapeDtypeStruct((2, PROWS, SUB), jnp.int32),
            jax.ShapeDtypeStruct((PROWS, SUB), f32),
        ],
    )(edge_index, edge_weight.reshape(EROWS, SUB))

    prm = []
    for l in range(2):
        cf = -0.5 / (EPS + sigma[l, 0, 0] ** 2)
        prm.append(jnp.concatenate([mu[l, 0, 0][None], cf[None],
                                    jnp.zeros((14,), f32)]))

    b1r = b1.reshape(1, H)
    cb0 = conv_bias[0].reshape(1, H)
    cb1 = conv_bias[1].reshape(1, H)
    f0 = fuse[0].reshape(1, 1)
    f1 = fuse[1].reshape(1, 1)
    bo = bout.reshape(1, C)

    nf = jax.ShapeDtypeStruct((N, H), f32)
    h, xa1, hr1 = pl.pallas_call(
        _tc1_body,
        grid=(GRID,),
        in_specs=[
            pl.BlockSpec((RB, F_IN), lambda i: (i, 0)),
            _full((F_IN, H)), _full((1, H)), _full((H, H)), _full((H, H)),
        ],
        out_specs=[
            pl.BlockSpec((RB, H), lambda i: (i, 0)),
            pl.BlockSpec((RB, 48), lambda i: (i, 0)),
            pl.BlockSpec((RB, H), lambda i: (i, 0)),
        ],
        out_shape=[nf, jax.ShapeDtypeStruct((N, 48), f32), nf],
    )(x, W1, b1r, gW[0], rootW[0])

    zr48 = jnp.zeros((ZROWS, 48), f32)
    p1 = _sc_segment(48, 512)(xa1, ipk, ew2, prm[0], zr48)

    h2, xa2, hr2, deg8 = pl.pallas_call(
        _tc2_body,
        grid=(GRID,),
        in_specs=[
            pl.BlockSpec((2, RB, 48), lambda i: (0, i, 0)),
            pl.BlockSpec((RB, H), lambda i: (i, 0)),
            pl.BlockSpec((RB, H), lambda i: (i, 0)),
            _full((1, H)), _full((1, 1)), _full((H, H)), _full((H, H)),
        ],
        out_specs=[
            pl.BlockSpec((RB, H), lambda i: (i, 0)),
            pl.BlockSpec((RB, H), lambda i: (i, 0)),
            pl.BlockSpec((RB, H), lambda i: (i, 0)),
            pl.BlockSpec((RB, 8), lambda i: (i, 0)),
        ],
        out_shape=[nf, nf, nf, jax.ShapeDtypeStruct((N, 8), f32)],
    )(p1, h, hr1, cb0, f0, gW[1], rootW[1])

    zr32 = jnp.zeros((ZROWS, H), f32)
    p2 = _sc_segment(H, 1024)(xa2, ipk, ew2, prm[1], zr32)

    out8 = pl.pallas_call(
        _tc3_body,
        grid=(GRID,),
        in_specs=[
            pl.BlockSpec((2, RB, H), lambda i: (0, i, 0)),
            pl.BlockSpec((RB, 8), lambda i: (i, 0)),
            pl.BlockSpec((RB, H), lambda i: (i, 0)),
            pl.BlockSpec((RB, H), lambda i: (i, 0)),
            _full((1, H)), _full((1, 1)), _full((H, C)), _full((1, C)),
        ],
        out_specs=[pl.BlockSpec((RB, 8), lambda i: (i, 0))],
        out_shape=[jax.ShapeDtypeStruct((N, 8), f32)],
    )(p2, deg8, h2, hr2, cb1, f1, Wout, bo)[0]

    return out8[:, :C]


# speedup vs baseline: 15.7814x; 1.0146x over previous
import functools

import jax
import jax.numpy as jnp
from jax import lax
from jax.experimental import pallas as pl
from jax.experimental.pallas import tpu as pltpu
from jax.experimental.pallas import tpu_sc as plsc

N = 10000
E = 320000
F_IN = 128
H = 32
C = 2
EPS = 1e-15

NW = 32
NSUB = 16
CHUNK = 1024
SUB = 128
NSUBC = CHUNK // SUB
EPT = 10240
E_PAD = NW * EPT
N_CHUNKS = EPT // CHUNK
ACC_ROWS = 10240
ZROWS = ACC_ROWS // NSUB

RB = 2000
GRID = N // RB

EROWS = E // SUB
PROWS = E_PAD // SUB
PB = 20
PGRID = PROWS // PB
PREAL = EROWS // PB


RPT = EPT // SUB


def _sc_segment(width, chunk):
    nsubc = chunk // SUB
    n_chunks = EPT // chunk
    mesh = plsc.VectorSubcoreMesh(core_axis_name="c", subcore_axis_name="s")

    def body(table, ipk, ew2, prm, zrows, out,
             acc, ipack, ewv, gv, rows, prmv, gs0, gs1, ss0, ss1):
        c = lax.axis_index("c")
        s = lax.axis_index("s")
        wid = c * NSUB + s
        gsem = [gs0, gs1]
        ssem = [ss0, ss1]
        pltpu.sync_copy(ipk.at[pl.ds(wid * 2 * RPT, 2 * RPT)], ipack)
        pltpu.sync_copy(ew2.at[pl.ds(wid * RPT, RPT)], ewv)

        descs_g = [None, None]
        descs_s = [None, None]

        def fire_gathers(k):
            b = k % 2
            descs_g[b] = [
                pltpu.async_copy(table.at[ipack.at[2 * (k * nsubc + j)]],
                                 rows.at[b, pl.ds(j * SUB, SUB)], gsem[b])
                for j in range(nsubc)
            ]

        fire_gathers(0)
        pltpu.sync_copy(zrows, acc.at[pl.ds(s * ZROWS, ZROWS)])
        pltpu.sync_copy(prm, prmv)
        pv = prmv[...]
        mu = pv[0]
        cf = pv[1]

        def gbody(i, _):
            w = ewv[i >> 3, pl.ds((i & 7) * 16, 16)]
            d = w - mu
            gv[pl.ds(i * 16, 16)] = jnp.exp(d * d * cf)
            return _
        lax.fori_loop(0, EPT // 16, gbody, 0)
        plsc.subcore_barrier()

        for k in range(n_chunks):
            b = k % 2
            if k + 1 < n_chunks:
                b1 = (k + 1) % 2
                if descs_s[b1] is not None:
                    for dd in descs_s[b1]:
                        dd.wait()
                fire_gathers(k + 1)
            for dd in descs_g[b]:
                dd.wait()

            def sbody(bb, _):
                g16 = gv[pl.ds(k * chunk + bb * 16, 16)]
                e0 = bb * 16
                for j in range(16):
                    ge = g16[j]
                    rows[b, e0 + j, pl.ds(0, 16)] = (
                        rows[b, e0 + j, pl.ds(0, 16)] * ge)
                    rows[b, e0 + j, pl.ds(16, 16)] = (
                        rows[b, e0 + j, pl.ds(16, 16)] * ge)
                return _
            lax.fori_loop(0, chunk // 16, sbody, 0)

            descs_s[b] = [
                pltpu.async_copy(rows.at[b, pl.ds(j * SUB, SUB)],
                                 acc.at[ipack.at[2 * (k * nsubc + j) + 1]],
                                 ssem[b], add=True)
                for j in range(nsubc)
            ]
        for b in range(2):
            if descs_s[b] is not None:
                for dd in descs_s[b]:
                    dd.wait()
        plsc.subcore_barrier()
        pltpu.sync_copy(acc.at[pl.ds(s * ZROWS, ZROWS)],
                        out.at[c, pl.ds(s * ZROWS, ZROWS)])

    return pl.kernel(
        body,
        out_type=jax.ShapeDtypeStruct((2, ACC_ROWS, width), jnp.float32),
        mesh=mesh,
        compiler_params=pltpu.CompilerParams(use_tc_tiling_on_sc=False),
        scratch_types=[
            pltpu.VMEM_SHARED((ACC_ROWS, width), jnp.float32),
            pltpu.VMEM((2 * RPT, SUB), jnp.int32),
            pltpu.VMEM((RPT, SUB), jnp.float32),
            pltpu.VMEM((EPT,), jnp.float32),
            pltpu.VMEM((2, chunk, width), jnp.float32),
            pltpu.VMEM((16,), jnp.float32),
            pltpu.SemaphoreType.DMA,
            pltpu.SemaphoreType.DMA,
            pltpu.SemaphoreType.DMA,
            pltpu.SemaphoreType.DMA,
        ],
    )


def _full(shape):
    return pl.BlockSpec(shape, lambda i: (0,) * len(shape))


def _prep_body(ei, ew, ipk_o, ew_o):
    npad = PROWS - EROWS
    row_i = lax.broadcasted_iota(jnp.int32, (npad, SUB), 0)
    lane_i = lax.broadcasted_iota(jnp.int32, (npad, SUB), 1)
    pad_i = row_i * SUB + lane_i
    srcp = pad_i % N
    dstp = N + pad_i % (ACC_ROWS - N)
    real = ei[...].reshape(2, EROWS, SUB)
    src_all = jnp.concatenate([real[0], srcp], axis=0)
    dst_all = jnp.concatenate([real[1], dstp], axis=0)
    ipk_o[...] = jnp.stack([src_all, dst_all], axis=1).reshape(2 * PROWS, SUB)
    ew_o[...] = jnp.concatenate(
        [ew[...], jnp.zeros((npad, SUB), jnp.float32)], axis=0)


def _tc1_body(x, W1, b1, gW0, rW0, hh_o, xa_o):
    h = jnp.maximum(
        jnp.dot(x[...], W1[...], preferred_element_type=jnp.float32) + b1[...],
        0.0)
    hr = jnp.dot(h, rW0[...], preferred_element_type=jnp.float32)
    hh_o[...] = jnp.concatenate([h, hr], axis=1)
    xg = jnp.dot(h, gW0[...], preferred_element_type=jnp.float32)
    extra = (lax.broadcasted_iota(jnp.int32, (RB, 16), 1) == 0).astype(jnp.float32)
    xa_o[...] = jnp.concatenate([xg, extra], axis=1)


def _tc2_body(p1, hh, cb0, f0, gW1, rW1, hh2_o, xa2_o):
    p = p1[...]
    su = p[0] + p[1]
    hhv = hh[...]
    deg = jnp.maximum(su[:, 32:33], 1.0)
    h2 = su[:, :32] / deg + hhv[:, 32:64] + cb0[...] + f0[0, 0] * hhv[:, :32]
    hr2 = jnp.dot(h2, rW1[...], preferred_element_type=jnp.float32)
    hh2_o[...] = jnp.concatenate(
        [h2, hr2, jnp.broadcast_to(deg, (RB, 16))], axis=1)
    xa2_o[...] = jnp.dot(h2, gW1[...], preferred_element_type=jnp.float32)


def _tc3_body(p2, hh2, cb1, f1, Wo, bo, out_o):
    p = p2[...]
    su = p[0] + p[1]
    hhv = hh2[...]
    deg = hhv[:, 64:65]
    h3 = su / deg + hhv[:, 32:64] + cb1[...] + f1[0, 0] * hhv[:, :32]
    WoP = jnp.concatenate([Wo[...], jnp.zeros((H, 8 - C), jnp.float32)], 1)
    boP = jnp.concatenate([bo[...], jnp.zeros((1, 8 - C), jnp.float32)], 1)
    lg = jnp.dot(h3, WoP, preferred_element_type=jnp.float32) + boP
    mask = lax.broadcasted_iota(jnp.int32, (RB, 8), 1) < C
    m = jnp.max(jnp.where(mask, lg, -1e30), axis=1, keepdims=True)
    ex = jnp.where(mask, jnp.exp(lg - m), 0.0)
    lse = m + jnp.log(jnp.sum(ex, axis=1, keepdims=True))
    out_o[...] = (lg - lse)[:, :C]


def kernel(x, edge_index, edge_weight, W1, b1, gW, rootW, mu, sigma,
           conv_bias, fuse, Wout, bout):
    f32 = jnp.float32
    ipk, ew2 = pl.pallas_call(
        _prep_body,
        out_shape=[
            jax.ShapeDtypeStruct((2 * PROWS, SUB), jnp.int32),
            jax.ShapeDtypeStruct((PROWS, SUB), f32),
        ],
    )(edge_index, edge_weight.reshape(EROWS, SUB))

    prm = []
    for l in range(2):
        cf = -0.5 / (EPS + sigma[l, 0, 0] ** 2)
        prm.append(jnp.concatenate([mu[l, 0, 0][None], cf[None],
                                    jnp.zeros((14,), f32)]))

    b1r = b1.reshape(1, H)
    cb0 = conv_bias[0].reshape(1, H)
    cb1 = conv_bias[1].reshape(1, H)
    f0 = fuse[0].reshape(1, 1)
    f1 = fuse[1].reshape(1, 1)
    bo = bout.reshape(1, C)

    hh1, xa1 = pl.pallas_call(
        _tc1_body,
        grid=(GRID,),
        in_specs=[
            pl.BlockSpec((RB, F_IN), lambda i: (i, 0)),
            _full((F_IN, H)), _full((1, H)), _full((H, H)), _full((H, H)),
        ],
        out_specs=[
            pl.BlockSpec((RB, 64), lambda i: (i, 0)),
            pl.BlockSpec((RB, 48), lambda i: (i, 0)),
        ],
        out_shape=[
            jax.ShapeDtypeStruct((N, 64), f32),
            jax.ShapeDtypeStruct((N, 48), f32),
        ],
    )(x, W1, b1r, gW[0], rootW[0])

    zr48 = jnp.zeros((ZROWS, 48), f32)
    p1 = _sc_segment(48, 512)(xa1, ipk, ew2, prm[0], zr48)

    hh2, xa2 = pl.pallas_call(
        _tc2_body,
        grid=(GRID,),
        in_specs=[
            pl.BlockSpec((2, RB, 48), lambda i: (0, i, 0)),
            pl.BlockSpec((RB, 64), lambda i: (i, 0)),
            _full((1, H)), _full((1, 1)), _full((H, H)), _full((H, H)),
        ],
        out_specs=[
            pl.BlockSpec((RB, 80), lambda i: (i, 0)),
            pl.BlockSpec((RB, H), lambda i: (i, 0)),
        ],
        out_shape=[
            jax.ShapeDtypeStruct((N, 80), f32),
            jax.ShapeDtypeStruct((N, H), f32),
        ],
    )(p1, hh1, cb0, f0, gW[1], rootW[1])

    zr32 = jnp.zeros((ZROWS, H), f32)
    p2 = _sc_segment(H, 1024)(xa2, ipk, ew2, prm[1], zr32)

    return pl.pallas_call(
        _tc3_body,
        grid=(GRID,),
        in_specs=[
            pl.BlockSpec((2, RB, H), lambda i: (0, i, 0)),
            pl.BlockSpec((RB, 80), lambda i: (i, 0)),
            _full((1, H)), _full((1, 1)), _full((H, C)), _full((1, C)),
        ],
        out_specs=[pl.BlockSpec((RB, C), lambda i: (i, 0))],
        out_shape=[jax.ShapeDtypeStruct((N, C), f32)],
    )(p2, hh2, cb1, f1, Wout, bo)[0]


# speedup vs baseline: 15.8835x; 1.0065x over previous
import functools

import jax
import jax.numpy as jnp
from jax import lax
from jax.experimental import pallas as pl
from jax.experimental.pallas import tpu as pltpu
from jax.experimental.pallas import tpu_sc as plsc

N = 10000
E = 320000
F_IN = 128
H = 32
C = 2
EPS = 1e-15

NW = 32
NSUB = 16
CHUNK = 1024
SUB = 128
NSUBC = CHUNK // SUB
EPT = 10240
E_PAD = NW * EPT
N_CHUNKS = EPT // CHUNK
ACC_ROWS = 10240
ZROWS = ACC_ROWS // NSUB

RB = 2000
GRID = N // RB

EROWS = E // SUB
PROWS = E_PAD // SUB
PB = 20
PGRID = PROWS // PB
PREAL = EROWS // PB


RPT = EPT // SUB


def _sc_segment(width, chunk, NBUF):
    nsubc = chunk // SUB
    n_chunks = EPT // chunk
    mesh = plsc.VectorSubcoreMesh(core_axis_name="c", subcore_axis_name="s")

    def body(table, ipk, ew2, prm, zrows, out,
             acc, ipack, ewv, rows, prmv, *sems):
        c = lax.axis_index("c")
        s = lax.axis_index("s")
        wid = c * NSUB + s
        gsem = list(sems[:NBUF])
        ssem = list(sems[NBUF:])
        pltpu.sync_copy(ipk.at[pl.ds(wid * 2 * RPT, 2 * RPT)], ipack)
        pltpu.sync_copy(ew2.at[pl.ds(wid * RPT, RPT)], ewv)

        descs_g = [None] * NBUF
        descs_s = [None] * NBUF

        def fire_gathers(k):
            b = k % NBUF
            descs_g[b] = [
                pltpu.async_copy(table.at[ipack.at[2 * (k * nsubc + j)]],
                                 rows.at[b, pl.ds(j * SUB, SUB)], gsem[b])
                for j in range(nsubc)
            ]

        fire_gathers(0)
        pltpu.sync_copy(zrows, acc.at[pl.ds(s * ZROWS, ZROWS)])
        pltpu.sync_copy(prm, prmv)
        pv = prmv[...]
        mu = pv[0]
        cf = pv[1]

        def gbody(i, _):
            w = ewv[i >> 3, pl.ds((i & 7) * 16, 16)]
            d = w - mu
            ewv[i >> 3, pl.ds((i & 7) * 16, 16)] = jnp.exp(d * d * cf)
            return _
        lax.fori_loop(0, EPT // 16, gbody, 0)
        plsc.subcore_barrier()

        for k in range(n_chunks):
            b = k % NBUF
            if k + 1 < n_chunks:
                b1 = (k + 1) % NBUF
                if descs_s[b1] is not None:
                    for dd in descs_s[b1]:
                        dd.wait()
                fire_gathers(k + 1)
            for dd in descs_g[b]:
                dd.wait()

            def sbody(bb, _):
                e0 = k * chunk + bb * 16
                g16 = ewv[e0 >> 7, pl.ds(e0 & 127, 16)]
                e0 = bb * 16
                for j in range(16):
                    ge = g16[j]
                    rows[b, e0 + j, pl.ds(0, 16)] = (
                        rows[b, e0 + j, pl.ds(0, 16)] * ge)
                    rows[b, e0 + j, pl.ds(16, 16)] = (
                        rows[b, e0 + j, pl.ds(16, 16)] * ge)
                return _
            lax.fori_loop(0, chunk // 16, sbody, 0)

            descs_s[b] = [
                pltpu.async_copy(rows.at[b, pl.ds(j * SUB, SUB)],
                                 acc.at[ipack.at[2 * (k * nsubc + j) + 1]],
                                 ssem[b], add=True)
                for j in range(nsubc)
            ]
        for b in range(NBUF):
            if descs_s[b] is not None:
                for dd in descs_s[b]:
                    dd.wait()
        plsc.subcore_barrier()
        pltpu.sync_copy(acc.at[pl.ds(s * ZROWS, ZROWS)],
                        out.at[c, pl.ds(s * ZROWS, ZROWS)])

    return pl.kernel(
        body,
        out_type=jax.ShapeDtypeStruct((2, ACC_ROWS, width), jnp.float32),
        mesh=mesh,
        compiler_params=pltpu.CompilerParams(use_tc_tiling_on_sc=False),
        scratch_types=[
            pltpu.VMEM_SHARED((ACC_ROWS, width), jnp.float32),
            pltpu.VMEM((2 * RPT, SUB), jnp.int32),
            pltpu.VMEM((RPT, SUB), jnp.float32),
            pltpu.VMEM((NBUF, chunk, width), jnp.float32),
            pltpu.VMEM((16,), jnp.float32),
        ] + [pltpu.SemaphoreType.DMA] * (2 * NBUF),
    )


def _full(shape):
    return pl.BlockSpec(shape, lambda i: (0,) * len(shape))


def _prep_body(ei, ew, ipk_o, ew_o):
    npad = PROWS - EROWS
    row_i = lax.broadcasted_iota(jnp.int32, (npad, SUB), 0)
    lane_i = lax.broadcasted_iota(jnp.int32, (npad, SUB), 1)
    pad_i = row_i * SUB + lane_i
    srcp = pad_i % N
    dstp = N + pad_i % (ACC_ROWS - N)
    real = ei[...].reshape(2, EROWS, SUB)
    src_all = jnp.concatenate([real[0], srcp], axis=0)
    dst_all = jnp.concatenate([real[1], dstp], axis=0)
    ipk_o[...] = jnp.stack([src_all, dst_all], axis=1).reshape(2 * PROWS, SUB)
    ew_o[...] = jnp.concatenate(
        [ew[...], jnp.zeros((npad, SUB), jnp.float32)], axis=0)


def _tc1_body(x, W1, b1, gW0, rW0, hh_o, xa_o):
    h = jnp.maximum(
        jnp.dot(x[...], W1[...], preferred_element_type=jnp.float32) + b1[...],
        0.0)
    hr = jnp.dot(h, rW0[...], preferred_element_type=jnp.float32)
    hh_o[...] = jnp.concatenate([h, hr], axis=1)
    xg = jnp.dot(h, gW0[...], preferred_element_type=jnp.float32)
    extra = (lax.broadcasted_iota(jnp.int32, (RB, 16), 1) == 0).astype(jnp.float32)
    xa_o[...] = jnp.concatenate([xg, extra], axis=1)


def _tc2_body(p1, hh, cb0, f0, gW1, rW1, hh2_o, xa2_o):
    p = p1[...]
    su = p[0] + p[1]
    hhv = hh[...]
    deg = jnp.maximum(su[:, 32:33], 1.0)
    h2 = su[:, :32] / deg + hhv[:, 32:64] + cb0[...] + f0[0, 0] * hhv[:, :32]
    hr2 = jnp.dot(h2, rW1[...], preferred_element_type=jnp.float32)
    hh2_o[...] = jnp.concatenate(
        [h2, hr2, jnp.broadcast_to(deg, (RB, 16))], axis=1)
    xa2_o[...] = jnp.dot(h2, gW1[...], preferred_element_type=jnp.float32)


def _tc3_body(p2, hh2, cb1, f1, Wo, bo, out_o):
    p = p2[...]
    su = p[0] + p[1]
    hhv = hh2[...]
    deg = hhv[:, 64:65]
    h3 = su / deg + hhv[:, 32:64] + cb1[...] + f1[0, 0] * hhv[:, :32]
    WoP = jnp.concatenate([Wo[...], jnp.zeros((H, 8 - C), jnp.float32)], 1)
    boP = jnp.concatenate([bo[...], jnp.zeros((1, 8 - C), jnp.float32)], 1)
    lg = jnp.dot(h3, WoP, preferred_element_type=jnp.float32) + boP
    mask = lax.broadcasted_iota(jnp.int32, (RB, 8), 1) < C
    m = jnp.max(jnp.where(mask, lg, -1e30), axis=1, keepdims=True)
    ex = jnp.where(mask, jnp.exp(lg - m), 0.0)
    lse = m + jnp.log(jnp.sum(ex, axis=1, keepdims=True))
    out_o[...] = (lg - lse)[:, :C]


def kernel(x, edge_index, edge_weight, W1, b1, gW, rootW, mu, sigma,
           conv_bias, fuse, Wout, bout):
    f32 = jnp.float32
    ipk, ew2 = pl.pallas_call(
        _prep_body,
        out_shape=[
            jax.ShapeDtypeStruct((2 * PROWS, SUB), jnp.int32),
            jax.ShapeDtypeStruct((PROWS, SUB), f32),
        ],
    )(edge_index, edge_weight.reshape(EROWS, SUB))

    prm = []
    for l in range(2):
        cf = -0.5 / (EPS + sigma[l, 0, 0] ** 2)
        prm.append(jnp.concatenate([mu[l, 0, 0][None], cf[None],
                                    jnp.zeros((14,), f32)]))

    b1r = b1.reshape(1, H)
    cb0 = conv_bias[0].reshape(1, H)
    cb1 = conv_bias[1].reshape(1, H)
    f0 = fuse[0].reshape(1, 1)
    f1 = fuse[1].reshape(1, 1)
    bo = bout.reshape(1, C)

    hh1, xa1 = pl.pallas_call(
        _tc1_body,
        grid=(GRID,),
        in_specs=[
            pl.BlockSpec((RB, F_IN), lambda i: (i, 0)),
            _full((F_IN, H)), _full((1, H)), _full((H, H)), _full((H, H)),
        ],
        out_specs=[
            pl.BlockSpec((RB, 64), lambda i: (i, 0)),
            pl.BlockSpec((RB, 48), lambda i: (i, 0)),
        ],
        out_shape=[
            jax.ShapeDtypeStruct((N, 64), f32),
            jax.ShapeDtypeStruct((N, 48), f32),
        ],
    )(x, W1, b1r, gW[0], rootW[0])

    zr48 = jnp.zeros((ZROWS, 48), f32)
    p1 = _sc_segment(48, 512, 2)(xa1, ipk, ew2, prm[0], zr48)

    hh2, xa2 = pl.pallas_call(
        _tc2_body,
        grid=(GRID,),
        in_specs=[
            pl.BlockSpec((2, RB, 48), lambda i: (0, i, 0)),
            pl.BlockSpec((RB, 64), lambda i: (i, 0)),
            _full((1, H)), _full((1, 1)), _full((H, H)), _full((H, H)),
        ],
        out_specs=[
            pl.BlockSpec((RB, 80), lambda i: (i, 0)),
            pl.BlockSpec((RB, H), lambda i: (i, 0)),
        ],
        out_shape=[
            jax.ShapeDtypeStruct((N, 80), f32),
            jax.ShapeDtypeStruct((N, H), f32),
        ],
    )(p1, hh1, cb0, f0, gW[1], rootW[1])

    zr32 = jnp.zeros((ZROWS, H), f32)
    p2 = _sc_segment(H, 512, 3)(xa2, ipk, ew2, prm[1], zr32)

    return pl.pallas_call(
        _tc3_body,
        grid=(GRID,),
        in_specs=[
            pl.BlockSpec((2, RB, H), lambda i: (0, i, 0)),
            pl.BlockSpec((RB, 80), lambda i: (i, 0)),
            _full((1, H)), _full((1, 1)), _full((H, C)), _full((1, C)),
        ],
        out_specs=[pl.BlockSpec((RB, C), lambda i: (i, 0))],
        out_shape=[jax.ShapeDtypeStruct((N, C), f32)],
    )(p2, hh2, cb1, f1, Wout, bo)[0]
